# Initial kernel scaffold; baseline (speedup 1.0000x reference)
import jax
import jax.numpy as jnp
from jax.experimental import pallas as pl
from jax.experimental.pallas import tpu as pltpu

_B, _N, _M = 4, 8192, 2048
_C1, _C2 = 64, 128
_COUT = 128
_TN = 256
_NB = _N // _TN
_TN2 = 1024
_NB2 = _N // _TN2

_HI = jax.lax.Precision.HIGHEST


def _fp_main(kc_ref, uc_ref, kf_ref, uf_ref, w0_ref, x_ref, sums_ref, g_ref):
    b = pl.program_id(0)
    nb = pl.program_id(1)

    @pl.when(jnp.logical_and(b == 0, nb == 0))
    def _():
        sums_ref[...] = jnp.zeros_like(sums_ref)

    @pl.when(nb == 0)
    def _():
        g = jnp.dot(w0_ref[:, :_C2], kf_ref[0],
                    preferred_element_type=jnp.float32, precision=_HI)
        g_ref[...] = g.astype(jnp.bfloat16)

    kc = kc_ref[0]
    uc = uc_ref[0]
    dx = kc[:, 0:1] - uc[0:1, :]
    dy = kc[:, 1:2] - uc[1:2, :]
    dz = kc[:, 2:3] - uc[2:3, :]
    d2 = dx * dx + dy * dy + dz * dz

    inf = jnp.float32(jnp.inf)
    m1 = jnp.min(d2, axis=0, keepdims=True)
    d2a = jnp.where(d2 <= m1, inf, d2)
    m2 = jnp.min(d2a, axis=0, keepdims=True)
    d2b = jnp.where(d2a <= m2, inf, d2a)
    m3 = jnp.min(d2b, axis=0, keepdims=True)

    r1 = 1.0 / (m1 + 1e-8)
    r2 = 1.0 / (m2 + 1e-8)
    r3 = 1.0 / (m3 + 1e-8)
    norm = r1 + r2 + r3
    w1 = (r1 / norm).astype(jnp.bfloat16)
    w2 = (r2 / norm).astype(jnp.bfloat16)
    w3 = (r3 / norm).astype(jnp.bfloat16)

    zero = jnp.zeros_like(w1)
    s = jnp.where(d2 == m1, w1,
                  jnp.where(d2 == m2, w2,
                            jnp.where(d2 == m3, w3, zero)))

    x = jnp.dot(g_ref[...], s, preferred_element_type=jnp.float32)
    x = x + jnp.dot(w0_ref[:, _C2:], uf_ref[0],
                    preferred_element_type=jnp.float32, precision=_HI)
    x_ref[0] = x

    sums_ref[:, 0:1] += jnp.sum(x, axis=1, keepdims=True)
    sums_ref[:, 1:2] += jnp.sum(x * x, axis=1, keepdims=True)


def _fp_norm(x_ref, sums_ref, gm_ref, bt_ref, o_ref):
    cnt = jnp.float32(_B * _N)
    mean = sums_ref[:, 0:1] / cnt
    var = sums_ref[:, 1:2] / cnt - mean * mean
    inv = jax.lax.rsqrt(var + 1e-5)
    scale = gm_ref[...] * inv
    shift = bt_ref[...] - mean * scale
    o_ref[0] = jnp.maximum(x_ref[0] * scale + shift, 0.0)


def kernel(unknown, known, unknow_feats, known_feats, W0, gamma0, beta0):
    uc = jnp.concatenate(
        [jnp.swapaxes(unknown, 1, 2),
         jnp.zeros((_B, 5, _N), jnp.float32)], axis=1)
    kc = jnp.concatenate(
        [known, jnp.zeros((_B, _M, 5), jnp.float32)], axis=2)

    x_pre, sums = pl.pallas_call(
        _fp_main,
        grid=(_B, _NB),
        in_specs=[
            pl.BlockSpec((1, _M, 8), lambda b, n: (b, 0, 0)),
            pl.BlockSpec((1, 8, _TN), lambda b, n: (b, 0, n)),
            pl.BlockSpec((1, _C2, _M), lambda b, n: (b, 0, 0)),
            pl.BlockSpec((1, _C1, _TN), lambda b, n: (b, 0, n)),
            pl.BlockSpec((_COUT, _C1 + _C2), lambda b, n: (0, 0)),
        ],
        out_specs=[
            pl.BlockSpec((1, _COUT, _TN), lambda b, n: (b, 0, n)),
            pl.BlockSpec((_COUT, 8), lambda b, n: (0, 0)),
        ],
        out_shape=[
            jax.ShapeDtypeStruct((_B, _COUT, _N), jnp.float32),
            jax.ShapeDtypeStruct((_COUT, 8), jnp.float32),
        ],
        scratch_shapes=[pltpu.VMEM((_COUT, _M), jnp.bfloat16)],
        compiler_params=pltpu.CompilerParams(
            dimension_semantics=("arbitrary", "arbitrary")),
    )(kc, uc, known_feats, unknow_feats, W0)

    out = pl.pallas_call(
        _fp_norm,
        grid=(_B, _NB2),
        in_specs=[
            pl.BlockSpec((1, _COUT, _TN2), lambda b, n: (b, 0, n)),
            pl.BlockSpec((_COUT, 8), lambda b, n: (0, 0)),
            pl.BlockSpec((_COUT, 1), lambda b, n: (0, 0)),
            pl.BlockSpec((_COUT, 1), lambda b, n: (0, 0)),
        ],
        out_specs=pl.BlockSpec((1, _COUT, _TN2), lambda b, n: (b, 0, n)),
        out_shape=jax.ShapeDtypeStruct((_B, _COUT, _N), jnp.float32),
        compiler_params=pltpu.CompilerParams(
            dimension_semantics=("arbitrary", "arbitrary")),
    )(x_pre, sums, gamma0.reshape(_COUT, 1), beta0.reshape(_COUT, 1))

    return out


# speedup vs baseline: 29.0964x; 29.0964x over previous
import jax
import jax.numpy as jnp
from jax.experimental import pallas as pl
from jax.experimental.pallas import tpu as pltpu

_B, _N, _M = 4, 8192, 2048
_C1, _C2 = 64, 128
_COUT = 128
_TN = 256
_NB = _N // _TN
_TN2 = 1024
_NB2 = _N // _TN2

_HI = jax.lax.Precision.HIGHEST


def _fp_main(kc_ref, uc_ref, kf_ref, uf_ref, w0_ref, x_ref, sums_ref, g_ref):
    b = pl.program_id(0)
    nb = pl.program_id(1)

    @pl.when(jnp.logical_and(b == 0, nb == 0))
    def _():
        sums_ref[...] = jnp.zeros_like(sums_ref)

    @pl.when(nb == 0)
    def _():
        g = jnp.dot(w0_ref[:, :_C2], kf_ref[0],
                    preferred_element_type=jnp.float32, precision=_HI)
        g_ref[...] = g.astype(jnp.bfloat16)

    kc = kc_ref[0]
    uc = uc_ref[0]
    inner = jnp.dot(kc.astype(jnp.bfloat16), uc.astype(jnp.bfloat16),
                    preferred_element_type=jnp.float32)
    k2 = kc[:, 0:1] * kc[:, 0:1] + kc[:, 1:2] * kc[:, 1:2] + kc[:, 2:3] * kc[:, 2:3]
    u2 = uc[0:1, :] * uc[0:1, :] + uc[1:2, :] * uc[1:2, :] + uc[2:3, :] * uc[2:3, :]
    d2 = jnp.maximum(u2 + k2 - 2.0 * inner, 0.0)

    inf = jnp.float32(jnp.inf)
    iota = jax.lax.broadcasted_iota(jnp.int32, (_M, _TN), 0)
    big = jnp.int32(_M)

    m1 = jnp.min(d2, axis=0, keepdims=True)
    i1 = jnp.min(jnp.where(d2 == m1, iota, big), axis=0, keepdims=True)
    d2a = jnp.where(iota == i1, inf, d2)
    m2 = jnp.min(d2a, axis=0, keepdims=True)
    i2 = jnp.min(jnp.where(d2a == m2, iota, big), axis=0, keepdims=True)
    d2b = jnp.where(iota == i2, inf, d2a)
    m3 = jnp.min(d2b, axis=0, keepdims=True)
    i3 = jnp.min(jnp.where(d2b == m3, iota, big), axis=0, keepdims=True)

    r1 = 1.0 / (m1 + 1e-8)
    r2 = 1.0 / (m2 + 1e-8)
    r3 = 1.0 / (m3 + 1e-8)
    norm = r1 + r2 + r3
    w1 = r1 / norm
    w2 = r2 / norm
    w3 = r3 / norm

    zero = jnp.zeros_like(w1)
    s = jnp.where(iota == i1, w1,
                  jnp.where(iota == i2, w2,
                            jnp.where(iota == i3, w3, zero)))
    s = s.astype(jnp.bfloat16)

    x = jnp.dot(g_ref[...], s, preferred_element_type=jnp.float32)
    x = x + jnp.dot(w0_ref[:, _C2:], uf_ref[0],
                    preferred_element_type=jnp.float32, precision=_HI)
    x_ref[0] = x

    sums_ref[:, 0:1] += jnp.sum(x, axis=1, keepdims=True)
    sums_ref[:, 1:2] += jnp.sum(x * x, axis=1, keepdims=True)


def _fp_norm(x_ref, sums_ref, gm_ref, bt_ref, o_ref):
    cnt = jnp.float32(_B * _N)
    mean = sums_ref[:, 0:1] / cnt
    var = sums_ref[:, 1:2] / cnt - mean * mean
    inv = jax.lax.rsqrt(var + 1e-5)
    scale = gm_ref[...] * inv
    shift = bt_ref[...] - mean * scale
    o_ref[0] = jnp.maximum(x_ref[0] * scale + shift, 0.0)


def kernel(unknown, known, unknow_feats, known_feats, W0, gamma0, beta0):
    uc = jnp.concatenate(
        [jnp.swapaxes(unknown, 1, 2),
         jnp.zeros((_B, 5, _N), jnp.float32)], axis=1)
    kc = jnp.concatenate(
        [known, jnp.zeros((_B, _M, 5), jnp.float32)], axis=2)

    x_pre, sums = pl.pallas_call(
        _fp_main,
        grid=(_B, _NB),
        in_specs=[
            pl.BlockSpec((1, _M, 8), lambda b, n: (b, 0, 0)),
            pl.BlockSpec((1, 8, _TN), lambda b, n: (b, 0, n)),
            pl.BlockSpec((1, _C2, _M), lambda b, n: (b, 0, 0)),
            pl.BlockSpec((1, _C1, _TN), lambda b, n: (b, 0, n)),
            pl.BlockSpec((_COUT, _C1 + _C2), lambda b, n: (0, 0)),
        ],
        out_specs=[
            pl.BlockSpec((1, _COUT, _TN), lambda b, n: (b, 0, n)),
            pl.BlockSpec((_COUT, 8), lambda b, n: (0, 0)),
        ],
        out_shape=[
            jax.ShapeDtypeStruct((_B, _COUT, _N), jnp.float32),
            jax.ShapeDtypeStruct((_COUT, 8), jnp.float32),
        ],
        scratch_shapes=[pltpu.VMEM((_COUT, _M), jnp.bfloat16)],
        compiler_params=pltpu.CompilerParams(
            dimension_semantics=("arbitrary", "arbitrary")),
    )(kc, uc, known_feats, unknow_feats, W0)

    out = pl.pallas_call(
        _fp_norm,
        grid=(_B, _NB2),
        in_specs=[
            pl.BlockSpec((1, _COUT, _TN2), lambda b, n: (b, 0, n)),
            pl.BlockSpec((_COUT, 8), lambda b, n: (0, 0)),
            pl.BlockSpec((_COUT, 1), lambda b, n: (0, 0)),
            pl.BlockSpec((_COUT, 1), lambda b, n: (0, 0)),
        ],
        out_specs=pl.BlockSpec((1, _COUT, _TN2), lambda b, n: (b, 0, n)),
        out_shape=jax.ShapeDtypeStruct((_B, _COUT, _N), jnp.float32),
        compiler_params=pltpu.CompilerParams(
            dimension_semantics=("arbitrary", "arbitrary")),
    )(x_pre, sums, gamma0.reshape(_COUT, 1), beta0.reshape(_COUT, 1))

    return out


# speedup vs baseline: 36.3324x; 1.2487x over previous
import jax
import jax.numpy as jnp
from jax.experimental import pallas as pl
from jax.experimental.pallas import tpu as pltpu

_B, _N, _M = 4, 8192, 2048
_C1, _C2 = 64, 128
_COUT = 128
_TN = 512
_NB = _N // _TN
_TN2 = 1024
_NB2 = _N // _TN2

_HI = jax.lax.Precision.HIGHEST


def _fp_main(kc_ref, uc_ref, kf_ref, uf_ref, w0_ref, x_ref, sums_ref, g_ref):
    b = pl.program_id(0)
    nb = pl.program_id(1)

    @pl.when(jnp.logical_and(b == 0, nb == 0))
    def _():
        sums_ref[...] = jnp.zeros_like(sums_ref)

    @pl.when(nb == 0)
    def _():
        g = jnp.dot(w0_ref[:, :_C2], kf_ref[0],
                    preferred_element_type=jnp.float32, precision=_HI)
        g_ref[...] = g.astype(jnp.bfloat16)

    kc = kc_ref[0]
    uc = uc_ref[0]
    inner = jnp.dot(kc.astype(jnp.bfloat16), uc.astype(jnp.bfloat16),
                    preferred_element_type=jnp.float32)
    k2 = kc[:, 0:1] * kc[:, 0:1] + kc[:, 1:2] * kc[:, 1:2] + kc[:, 2:3] * kc[:, 2:3]
    u2 = uc[0:1, :] * uc[0:1, :] + uc[1:2, :] * uc[1:2, :] + uc[2:3, :] * uc[2:3, :]
    d2 = jnp.maximum(u2 + k2 - 2.0 * inner, 0.0)

    inf = jnp.float32(jnp.inf)
    iota = jax.lax.broadcasted_iota(jnp.int32, (_M, _TN), 0)
    big = jnp.int32(_M)

    m1 = jnp.min(d2, axis=0, keepdims=True)
    i1 = jnp.min(jnp.where(d2 == m1, iota, big), axis=0, keepdims=True)
    d2a = jnp.where(iota == i1, inf, d2)
    m2 = jnp.min(d2a, axis=0, keepdims=True)
    i2 = jnp.min(jnp.where(d2a == m2, iota, big), axis=0, keepdims=True)
    d2b = jnp.where(iota == i2, inf, d2a)
    m3 = jnp.min(d2b, axis=0, keepdims=True)
    i3 = jnp.min(jnp.where(d2b == m3, iota, big), axis=0, keepdims=True)

    r1 = 1.0 / (m1 + 1e-8)
    r2 = 1.0 / (m2 + 1e-8)
    r3 = 1.0 / (m3 + 1e-8)
    norm = r1 + r2 + r3
    w1 = r1 / norm
    w2 = r2 / norm
    w3 = r3 / norm

    zero = jnp.zeros_like(w1)
    s = jnp.where(iota == i1, w1,
                  jnp.where(iota == i2, w2,
                            jnp.where(iota == i3, w3, zero)))
    s = s.astype(jnp.bfloat16)

    x = jnp.dot(g_ref[...], s, preferred_element_type=jnp.float32)
    x = x + jnp.dot(w0_ref[:, _C2:].astype(jnp.bfloat16),
                    uf_ref[0].astype(jnp.bfloat16),
                    preferred_element_type=jnp.float32)
    x_ref[0] = x

    sums_ref[:, 0:1] += jnp.sum(x, axis=1, keepdims=True)
    sums_ref[:, 1:2] += jnp.sum(x * x, axis=1, keepdims=True)


def _fp_norm(x_ref, sums_ref, gm_ref, bt_ref, o_ref):
    cnt = jnp.float32(_B * _N)
    mean = sums_ref[:, 0:1] / cnt
    var = sums_ref[:, 1:2] / cnt - mean * mean
    inv = jax.lax.rsqrt(var + 1e-5)
    scale = gm_ref[...] * inv
    shift = bt_ref[...] - mean * scale
    o_ref[0] = jnp.maximum(x_ref[0] * scale + shift, 0.0)


def kernel(unknown, known, unknow_feats, known_feats, W0, gamma0, beta0):
    uc = jnp.concatenate(
        [jnp.swapaxes(unknown, 1, 2),
         jnp.zeros((_B, 5, _N), jnp.float32)], axis=1)
    kc = jnp.concatenate(
        [known, jnp.zeros((_B, _M, 5), jnp.float32)], axis=2)

    x_pre, sums = pl.pallas_call(
        _fp_main,
        grid=(_B, _NB),
        in_specs=[
            pl.BlockSpec((1, _M, 8), lambda b, n: (b, 0, 0)),
            pl.BlockSpec((1, 8, _TN), lambda b, n: (b, 0, n)),
            pl.BlockSpec((1, _C2, _M), lambda b, n: (b, 0, 0)),
            pl.BlockSpec((1, _C1, _TN), lambda b, n: (b, 0, n)),
            pl.BlockSpec((_COUT, _C1 + _C2), lambda b, n: (0, 0)),
        ],
        out_specs=[
            pl.BlockSpec((1, _COUT, _TN), lambda b, n: (b, 0, n)),
            pl.BlockSpec((_COUT, 8), lambda b, n: (0, 0)),
        ],
        out_shape=[
            jax.ShapeDtypeStruct((_B, _COUT, _N), jnp.float32),
            jax.ShapeDtypeStruct((_COUT, 8), jnp.float32),
        ],
        scratch_shapes=[pltpu.VMEM((_COUT, _M), jnp.bfloat16)],
        compiler_params=pltpu.CompilerParams(
            dimension_semantics=("arbitrary", "arbitrary")),
    )(kc, uc, known_feats, unknow_feats, W0)

    out = pl.pallas_call(
        _fp_norm,
        grid=(_B, _NB2),
        in_specs=[
            pl.BlockSpec((1, _COUT, _TN2), lambda b, n: (b, 0, n)),
            pl.BlockSpec((_COUT, 8), lambda b, n: (0, 0)),
            pl.BlockSpec((_COUT, 1), lambda b, n: (0, 0)),
            pl.BlockSpec((_COUT, 1), lambda b, n: (0, 0)),
        ],
        out_specs=pl.BlockSpec((1, _COUT, _TN2), lambda b, n: (b, 0, n)),
        out_shape=jax.ShapeDtypeStruct((_B, _COUT, _N), jnp.float32),
        compiler_params=pltpu.CompilerParams(
            dimension_semantics=("arbitrary", "arbitrary")),
    )(x_pre, sums, gamma0.reshape(_COUT, 1), beta0.reshape(_COUT, 1))

    return out
